# baseline (speedup 1.0000x reference)
import functools

import jax
import jax.numpy as jnp
from jax import lax
from jax.experimental import pallas as pl
from jax.experimental.pallas import tpu as pltpu
from jax.experimental.pallas import tpu_sc as plsc

_N_EMBD = 768
_N_TOKEN = 77
_BATCH = 1024
_LANES = 16
_NUM_CORES = 2
_NUM_SUBCORES = 16
_NW = _NUM_CORES * _NUM_SUBCORES
_ROWS_PER_W = _BATCH // _NW
_TOK_PAD = 80
_CHUNK = 16
_N_CHUNKS = _TOK_PAD // _CHUNK


def _embed_body(tok_hbm, table_hbm, pos_hbm, out_hbm, pos_v, idx_v, buf_v):
    wid = lax.axis_index("subcore") * _NUM_CORES + lax.axis_index("core")
    base_row = wid * _ROWS_PER_W

    pltpu.sync_copy(pos_hbm, pos_v)

    @pl.loop(0, _ROWS_PER_W)
    def _(i):
        b = base_row + i
        for m in range(_N_CHUNKS):
            rows = min(_CHUNK, _N_TOKEN - m * _CHUNK)
            pltpu.sync_copy(
                tok_hbm.at[pl.ds(b * _TOK_PAD + m * _CHUNK, _CHUNK)], idx_v)
            pltpu.sync_copy(table_hbm.at[idx_v], buf_v)

            @pl.loop(0, rows)
            def _(r):
                @pl.loop(0, _N_EMBD, step=_LANES)
                def _(c):
                    sl = pl.ds(c, _LANES)
                    buf_v[r, sl] += pos_v[m * _CHUNK + r, sl]

            pltpu.sync_copy(
                buf_v.at[pl.ds(0, rows)],
                out_hbm.at[b, pl.ds(m * _CHUNK, rows)])


@jax.jit
def _embed(tokens, token_embedding, position_embedding):
    tok_pad = jnp.pad(tokens, ((0, 0), (0, _TOK_PAD - _N_TOKEN)))
    tok_flat = tok_pad.reshape(_BATCH * _TOK_PAD)
    mesh = plsc.VectorSubcoreMesh(
        core_axis_name="core", subcore_axis_name="subcore")
    kern = pl.kernel(
        _embed_body,
        out_type=jax.ShapeDtypeStruct((_BATCH, _N_TOKEN, _N_EMBD),
                                      jnp.float32),
        mesh=mesh,
        scratch_types=[
            pltpu.VMEM((_N_TOKEN, _N_EMBD), jnp.float32),
            pltpu.VMEM((_CHUNK,), jnp.int32),
            pltpu.VMEM((_CHUNK, _N_EMBD), jnp.float32),
        ],
        compiler_params=pltpu.CompilerParams(use_tc_tiling_on_sc=False),
    )
    return kern(tok_flat, token_embedding, position_embedding)


def kernel(tokens, token_embedding, position_embedding):
    return _embed(tokens, token_embedding, position_embedding)


# speedup vs baseline: 2.3298x; 2.3298x over previous
import jax
import jax.numpy as jnp
from jax import lax
from jax.experimental import pallas as pl
from jax.experimental.pallas import tpu as pltpu
from jax.experimental.pallas import tpu_sc as plsc

_N_EMBD = 768
_N_TOKEN = 77
_BATCH = 1024
_LANES = 16
_NUM_CORES = 2
_NUM_SUBCORES = 16
_NW = _NUM_CORES * _NUM_SUBCORES
_ROWS_PER_W = _BATCH // _NW
_TOK_PAD = 80
_CHUNK = 16
_NBUF = 5
_COL_OFF = (0, 16, 32, 48, 64)
_COL_ROWS = (16, 16, 16, 16, 13)


def _embed_body(tok_hbm, table_hbm, pos_hbm, out_hbm, pos_v, idx_v,
                b0, b1, b2, b3, b4, g0, g1, g2, g3, g4,
                w0, w1, w2, w3, w4):
    bufs = (b0, b1, b2, b3, b4)
    gsem = (g0, g1, g2, g3, g4)
    wsem = (w0, w1, w2, w3, w4)

    wid = lax.axis_index("subcore") * _NUM_CORES + lax.axis_index("core")
    base_row = wid * _ROWS_PER_W

    pltpu.sync_copy(pos_hbm, pos_v)
    pltpu.sync_copy(
        tok_hbm.at[pl.ds(base_row * _TOK_PAD, _ROWS_PER_W * _TOK_PAD)], idx_v)

    def g_pair(row, j):
        r = _COL_ROWS[j]
        loc = row * _TOK_PAD + _COL_OFF[j]
        src = table_hbm.at[idx_v.at[pl.ds(loc, r)]]
        return src, bufs[j].at[pl.ds(0, r)]

    def g_start(row, j):
        src, dst = g_pair(row, j)
        pltpu.async_copy(src, dst, gsem[j])

    def g_wait(row, j):
        src, dst = g_pair(row, j)
        pltpu.make_async_copy(src, dst, gsem[j]).wait()

    def w_pair(row, j):
        r = _COL_ROWS[j]
        return (bufs[j].at[pl.ds(0, r)],
                out_hbm.at[base_row + row, pl.ds(_COL_OFF[j], r)])

    def w_start(row, j):
        src, dst = w_pair(row, j)
        pltpu.async_copy(src, dst, wsem[j])

    def w_wait(row, j):
        src, dst = w_pair(row, j)
        pltpu.make_async_copy(src, dst, wsem[j]).wait()

    g_start(0, 0)
    g_start(0, 1)

    @pl.loop(0, _ROWS_PER_W)
    def _(i):
        for j in range(_NBUF):
            rows = _COL_ROWS[j]
            g_wait(i, j)

            if j < 3:
                jp = j + 2

                @pl.when(i >= 1)
                def _():
                    w_wait(i - 1, jp)

                g_start(i, jp)
            else:
                jp = j - 3
                w_wait(i, jp)

                @pl.when(i <= _ROWS_PER_W - 2)
                def _():
                    g_start(i + 1, jp)

            @pl.loop(0, rows)
            def _(r):
                prow = _COL_OFF[j] + r
                for c in range(0, _N_EMBD, _LANES):
                    sl = pl.ds(c, _LANES)
                    plsc.addupdate(bufs[j].at[r, sl], pos_v[prow, sl])

            w_start(i, j)

    for j in (2, 3, 4):
        w_wait(_ROWS_PER_W - 1, j)


@jax.jit
def _embed(tokens, token_embedding, position_embedding):
    tok_pad = jnp.pad(tokens, ((0, 0), (0, _TOK_PAD - _N_TOKEN)))
    tok_flat = tok_pad.reshape(_BATCH * _TOK_PAD)
    mesh = plsc.VectorSubcoreMesh(
        core_axis_name="core", subcore_axis_name="subcore")
    kern = pl.kernel(
        _embed_body,
        out_type=jax.ShapeDtypeStruct((_BATCH, _N_TOKEN, _N_EMBD),
                                      jnp.float32),
        mesh=mesh,
        scratch_types=[
            pltpu.VMEM((_N_TOKEN, _N_EMBD), jnp.float32),
            pltpu.VMEM((_ROWS_PER_W * _TOK_PAD,), jnp.int32),
        ]
        + [pltpu.VMEM((_CHUNK, _N_EMBD), jnp.float32) for _ in range(_NBUF)]
        + [pltpu.SemaphoreType.DMA for _ in range(2 * _NBUF)],
        compiler_params=pltpu.CompilerParams(use_tc_tiling_on_sc=False),
    )
    return kern(tok_flat, token_embedding, position_embedding)


def kernel(tokens, token_embedding, position_embedding):
    return _embed(tokens, token_embedding, position_embedding)


# speedup vs baseline: 3.3237x; 1.4266x over previous
import jax
import jax.numpy as jnp
from jax import lax
from jax.experimental import pallas as pl
from jax.experimental.pallas import tpu as pltpu
from jax.experimental.pallas import tpu_sc as plsc

_N_EMBD = 768
_N_TOKEN = 77
_BATCH = 1024
_LANES = 16
_NUM_CORES = 2
_NUM_SUBCORES = 16
_NW = _NUM_CORES * _NUM_SUBCORES
_ROWS_PER_W = _BATCH // _NW
_NBUF = 5
_COL_OFF = (0, 16, 32, 48, 64)
_COL_ROWS = (16, 16, 16, 16, 13)


def _embed_body(tok_hbm, table_hbm, pos_hbm, out_hbm, pos_v, idx_v,
                b0, b1, b2, b3, b4, g0, g1, g2, g3, g4,
                w0, w1, w2, w3, w4):
    bufs = (b0, b1, b2, b3, b4)
    gsem = (g0, g1, g2, g3, g4)
    wsem = (w0, w1, w2, w3, w4)

    wid = lax.axis_index("subcore") * _NUM_CORES + lax.axis_index("core")
    base_row = wid * _ROWS_PER_W

    pltpu.sync_copy(pos_hbm, pos_v)
    pltpu.sync_copy(tok_hbm.at[pl.ds(base_row, _ROWS_PER_W)], idx_v)

    def g_pair(row, j):
        src = table_hbm.at[idx_v.at[row, pl.ds(_COL_OFF[j], _COL_ROWS[j])]]
        return src, bufs[j]

    def g_start(row, j):
        src, dst = g_pair(row, j)
        pltpu.async_copy(src, dst, gsem[j])

    def g_wait(row, j):
        src, dst = g_pair(row, j)
        pltpu.make_async_copy(src, dst, gsem[j]).wait()

    def w_pair(row, j):
        return (bufs[j],
                out_hbm.at[base_row + row, pl.ds(_COL_OFF[j], _COL_ROWS[j])])

    def w_start(row, j):
        src, dst = w_pair(row, j)
        pltpu.async_copy(src, dst, wsem[j])

    def w_wait(row, j):
        src, dst = w_pair(row, j)
        pltpu.make_async_copy(src, dst, wsem[j]).wait()

    g_start(0, 0)
    g_start(0, 1)

    @pl.loop(0, _ROWS_PER_W)
    def _(i):
        for j in range(_NBUF):
            rows = _COL_ROWS[j]
            g_wait(i, j)

            if j < 3:
                jp = j + 2

                @pl.when(i >= 1)
                def _():
                    w_wait(i - 1, jp)

                g_start(i, jp)
            else:
                jp = j - 3
                w_wait(i, jp)

                @pl.when(i <= _ROWS_PER_W - 2)
                def _():
                    g_start(i + 1, jp)

            @pl.loop(0, rows)
            def _(r):
                prow = _COL_OFF[j] + r
                for c in range(0, _N_EMBD, _LANES):
                    sl = pl.ds(c, _LANES)
                    plsc.addupdate(bufs[j].at[r, sl], pos_v[prow, sl])

            w_start(i, j)

    for j in (2, 3, 4):
        w_wait(_ROWS_PER_W - 1, j)


@jax.jit
def _embed(tokens, token_embedding, position_embedding):
    mesh = plsc.VectorSubcoreMesh(
        core_axis_name="core", subcore_axis_name="subcore")
    kern = pl.kernel(
        _embed_body,
        out_type=jax.ShapeDtypeStruct((_BATCH, _N_TOKEN, _N_EMBD),
                                      jnp.float32),
        mesh=mesh,
        scratch_types=[
            pltpu.VMEM((_N_TOKEN, _N_EMBD), jnp.float32),
            pltpu.VMEM((_ROWS_PER_W, _N_TOKEN), jnp.int32),
        ]
        + [pltpu.VMEM((_COL_ROWS[j], _N_EMBD), jnp.float32)
           for j in range(_NBUF)]
        + [pltpu.SemaphoreType.DMA for _ in range(2 * _NBUF)],
    )
    return kern(tokens, token_embedding, position_embedding)


def kernel(tokens, token_embedding, position_embedding):
    return _embed(tokens, token_embedding, position_embedding)
